# baseline (speedup 1.0000x reference)
import functools

import jax
import jax.numpy as jnp
from jax import lax
from jax.experimental import pallas as pl
from jax.experimental.pallas import tpu as pltpu
from jax.experimental.pallas import tpu_sc as plsc

BATCH = 16384
EMBED = 32
SUBPACK = 8
HIDDEN = 256
OUT = 10

NC = 2
NS = 16
NW = NC * NS
HALF = BATCH // 2
BPW = HALF // NW
LANES = 16


def _gather_body(svc_hbm, loc_hbm, tim_hbm, ts_hbm, tl_hbm, tt_hbm,
                 out_s, out_l, out_t,
                 idx_v, x_v, sem):
    wid = lax.axis_index("s") * NC + lax.axis_index("c")
    base = wid * BPW
    lane = lax.iota(jnp.int32, LANES)
    zero = jnp.zeros((LANES,), jnp.int32)
    tables = ((svc_hbm, ts_hbm, out_s),
              (loc_hbm, tl_hbm, out_l),
              (tim_hbm, tt_hbm, out_t))
    for ih, th, oh in tables:
        pltpu.sync_copy(ih.at[pl.ds(base, BPW)], idx_v)

        def fire_body(g, _):
            iv = idx_v[pl.ds(g * LANES, LANES)]
            for k in range(LANES):
                sc = jnp.sum(jnp.where(lane == k, iv, zero))
                pltpu.async_copy(th.at[sc >> 3, sc & 7],
                                 x_v.at[g * LANES + k], sem)
            return 0

        lax.fori_loop(0, BPW // LANES, fire_body, 0)

        def drain_body(g, _):
            pltpu.make_async_copy(oh.at[pl.ds(base, LANES)],
                                  x_v.at[pl.ds(g * LANES, LANES)],
                                  sem).wait()
            return 0

        lax.fori_loop(0, BPW // LANES, drain_body, 0)
        pltpu.sync_copy(x_v, oh.at[pl.ds(base, BPW)])


_sc_gather = functools.partial(
    pl.kernel,
    out_type=[jax.ShapeDtypeStruct((HALF, EMBED), jnp.float32)] * 3,
    mesh=plsc.VectorSubcoreMesh(core_axis_name="c", subcore_axis_name="s"),
    scratch_types=[
        pltpu.VMEM((BPW,), jnp.int32),
        pltpu.VMEM((BPW, EMBED), jnp.float32),
        pltpu.SemaphoreType.DMA,
    ],
    compiler_params=pltpu.CompilerParams(needs_layout_passes=False),
)(_gather_body)


TILE = 8192


def _mlp_body(xs, xl, xt, w1, b1, w2, b2, w3, b3, out):
    x = jnp.concatenate([xs[...], xl[...], xt[...]], axis=-1)
    h = jnp.dot(x, w1[...], preferred_element_type=jnp.float32) + b1[...]
    h = jnp.maximum(h, 0.0)
    h = jnp.dot(h, w2[...], preferred_element_type=jnp.float32) + b2[...]
    h = jnp.maximum(h, 0.0)
    out[...] = jnp.dot(h, w3[...], preferred_element_type=jnp.float32) + b3[...]


def _mlp(xs, xl, xt, W1, b1, W2, b2, W3, b3):
    grid = HALF // TILE
    emb_spec = pl.BlockSpec((TILE, EMBED), lambda i: (i, 0))
    full = lambda a: pl.BlockSpec(a.shape, lambda i: (0,) * a.ndim)
    return pl.pallas_call(
        _mlp_body,
        grid=(grid,),
        in_specs=[emb_spec, emb_spec, emb_spec,
                  full(W1), full(b1), full(W2), full(b2), full(W3), full(b3)],
        out_specs=pl.BlockSpec((TILE, OUT), lambda i: (i, 0)),
        out_shape=jax.ShapeDtypeStruct((HALF, OUT), jnp.float32),
    )(xs, xl, xt, W1, b1, W2, b2, W3, b3)


def kernel(service_idx, location_idx, time_idx, T_service, T_location,
           T_time, W1, b1, W2, b2, W3, b3):
    svc = service_idx.astype(jnp.int32)
    loc = location_idx.astype(jnp.int32)
    tim = time_idx.astype(jnp.int32)
    ts8 = T_service.reshape(-1, SUBPACK, EMBED)
    tl8 = T_location.reshape(-1, SUBPACK, EMBED)
    tt8 = T_time.reshape(-1, SUBPACK, EMBED)
    b1r, b2r, b3r = (b1.reshape(1, HIDDEN), b2.reshape(1, HIDDEN),
                     b3.reshape(1, OUT))
    outs = []
    for h in range(2):
        sl = slice(h * HALF, (h + 1) * HALF)
        xs, xl, xt = _sc_gather(svc[sl], loc[sl], tim[sl], ts8, tl8, tt8)
        outs.append(_mlp(xs, xl, xt, W1, b1r, W2, b2r, W3, b3r))
    return jnp.concatenate(outs, axis=0)


# speedup vs baseline: 1.0158x; 1.0158x over previous
import functools

import jax
import jax.numpy as jnp
from jax import lax
from jax.experimental import pallas as pl
from jax.experimental.pallas import tpu as pltpu
from jax.experimental.pallas import tpu_sc as plsc

BATCH = 16384
EMBED = 32
SUBPACK = 8
HIDDEN = 256
OUT = 10

NC = 2
NS = 16
NW = NC * NS
BPW = BATCH // NW
LANES = 16


def _gather_body(svc_hbm, loc_hbm, tim_hbm, ts_hbm, tl_hbm, tt_hbm,
                 out_s, out_l, out_t,
                 idx_v, x_v, sem):
    wid = lax.axis_index("s") * NC + lax.axis_index("c")
    base = wid * BPW
    lane = lax.iota(jnp.int32, LANES)
    zero = jnp.zeros((LANES,), jnp.int32)
    tables = ((svc_hbm, ts_hbm, out_s),
              (loc_hbm, tl_hbm, out_l),
              (tim_hbm, tt_hbm, out_t))
    for ih, th, oh in tables:
        pltpu.sync_copy(ih.at[pl.ds(base, BPW)], idx_v)

        def fire_body(g, _):
            iv = idx_v[pl.ds(g * LANES, LANES)]
            for k in range(LANES):
                sc = jnp.sum(jnp.where(lane == k, iv, zero))
                pltpu.async_copy(th.at[sc >> 3, sc & 7],
                                 x_v.at[g * LANES + k], sem)
            return 0

        lax.fori_loop(0, BPW // LANES, fire_body, 0)

        def drain_body(g, _):
            pltpu.make_async_copy(oh.at[pl.ds(base, LANES)],
                                  x_v.at[pl.ds(g * LANES, LANES)],
                                  sem).wait()
            return 0

        lax.fori_loop(0, BPW // LANES, drain_body, 0)
        pltpu.sync_copy(x_v, oh.at[pl.ds(base, BPW)])


_sc_gather = functools.partial(
    pl.kernel,
    out_type=[jax.ShapeDtypeStruct((BATCH, EMBED), jnp.float32)] * 3,
    mesh=plsc.VectorSubcoreMesh(core_axis_name="c", subcore_axis_name="s"),
    scratch_types=[
        pltpu.VMEM((BPW,), jnp.int32),
        pltpu.VMEM((BPW, EMBED), jnp.float32),
        pltpu.SemaphoreType.DMA,
    ],
    compiler_params=pltpu.CompilerParams(needs_layout_passes=False),
)(_gather_body)


TILE = 8192


def _mlp_body(xs, xl, xt, w1, b1, w2, b2, w3, b3, out):
    x = jnp.concatenate([xs[...], xl[...], xt[...]], axis=-1)
    h = jnp.dot(x, w1[...], preferred_element_type=jnp.float32) + b1[...]
    h = jnp.maximum(h, 0.0)
    h = jnp.dot(h, w2[...], preferred_element_type=jnp.float32) + b2[...]
    h = jnp.maximum(h, 0.0)
    out[...] = jnp.dot(h, w3[...], preferred_element_type=jnp.float32) + b3[...]


def _mlp(xs, xl, xt, W1, b1, W2, b2, W3, b3):
    grid = BATCH // TILE
    emb_spec = pl.BlockSpec((TILE, EMBED), lambda i: (i, 0))
    full = lambda a: pl.BlockSpec(a.shape, lambda i: (0,) * a.ndim)
    return pl.pallas_call(
        _mlp_body,
        grid=(grid,),
        in_specs=[emb_spec, emb_spec, emb_spec,
                  full(W1), full(b1), full(W2), full(b2), full(W3), full(b3)],
        out_specs=pl.BlockSpec((TILE, OUT), lambda i: (i, 0)),
        out_shape=jax.ShapeDtypeStruct((BATCH, OUT), jnp.float32),
    )(xs, xl, xt, W1, b1, W2, b2, W3, b3)


def kernel(service_idx, location_idx, time_idx, T_service, T_location,
           T_time, W1, b1, W2, b2, W3, b3):
    svc = service_idx.astype(jnp.int32)
    loc = location_idx.astype(jnp.int32)
    tim = time_idx.astype(jnp.int32)
    ts8 = T_service.reshape(-1, SUBPACK, EMBED)
    tl8 = T_location.reshape(-1, SUBPACK, EMBED)
    tt8 = T_time.reshape(-1, SUBPACK, EMBED)
    xs, xl, xt = _sc_gather(svc, loc, tim, ts8, tl8, tt8)
    return _mlp(xs, xl, xt, W1,
                b1.reshape(1, HIDDEN), W2, b2.reshape(1, HIDDEN),
                W3, b3.reshape(1, OUT))
